# Initial kernel scaffold; baseline (speedup 1.0000x reference)
import jax
import jax.numpy as jnp
from jax.experimental import pallas as pl


def kernel(x, edge_index, batch, W1_l, W1_r, b1, W2_l, W2_r, b2, W3_l, W3_r, b3):
    raise NotImplementedError("write your pallas kernel here")


# speedup vs baseline: 2.7764x; 2.7764x over previous
import functools

import jax
import jax.numpy as jnp
from jax import lax
from jax.experimental import pallas as pl
from jax.experimental.pallas import tpu as pltpu
from jax.experimental.pallas import tpu_sc as plsc

N_NODES = 10000
NP = 10240
E_EDGES = 160000
EP = 163840
NW = 32
EPW = EP // NW
CH = 128
NCHUNK = EPW // CH
SL = NP // 16
TRASH = NP - 1


def _sc_agg(table, src_r, dst_r, D, n_pass, mult, want_deg):
  mesh = plsc.VectorSubcoreMesh(core_axis_name="c", subcore_axis_name="s")
  out_type = [jax.ShapeDtypeStruct((2, n_pass, NP, D), jnp.float32)]
  if want_deg:
    out_type.append(jax.ShapeDtypeStruct((2, NP, 16), jnp.float32))

  scratch = [
      pltpu.VMEM((NCHUNK, CH), jnp.int32),
      pltpu.VMEM((NCHUNK, CH), jnp.int32),
      pltpu.VMEM((CH,), jnp.int32),
      pltpu.VMEM((CH, D), jnp.float32),
      pltpu.VMEM((CH, D), jnp.float32),
      pltpu.VMEM((CH, 16), jnp.float32),
      pltpu.VMEM((CH, 16), jnp.float32),
      pltpu.VMEM_SHARED((NP, D), jnp.float32),
      pltpu.VMEM_SHARED((NP, 16), jnp.float32),
      pltpu.SemaphoreType.DMA,
  ]

  def body(table_h, src_h, dst_h, *refs):
    if want_deg:
      agg_out, deg_out = refs[0], refs[1]
      refs = refs[2:]
    else:
      agg_out = refs[0]
      refs = refs[1:]
    (src_v, dst_v, gidx, rows_v, zbuf, ones_v, z16, agg_sh, deg_sh,
     sem) = refs

    cid = lax.axis_index("c")
    sid = lax.axis_index("s")
    wid = cid * 16 + sid

    pltpu.sync_copy(src_h.at[wid], src_v)
    pltpu.sync_copy(dst_h.at[wid], dst_v)

    def fill(j, carry):
      for k in range(D // 16):
        zbuf[j, pl.ds(k * 16, 16)] = jnp.zeros((16,), jnp.float32)
      ones_v[j] = jnp.full((16,), 1.0, jnp.float32)
      z16[j] = jnp.zeros((16,), jnp.float32)
      return carry
    lax.fori_loop(0, CH, fill, 0)

    for cc in range(n_pass):
      for k in range(SL // CH):
        pltpu.sync_copy(zbuf, agg_sh.at[pl.ds(sid * SL + k * CH, CH)])
        if want_deg and cc == 0:
          pltpu.sync_copy(z16, deg_sh.at[pl.ds(sid * SL + k * CH, CH)])
      plsc.subcore_barrier()

      def chunk(j, carry):
        for k in range(CH // 16):
          sv = src_v[j, pl.ds(k * 16, 16)]
          gidx[pl.ds(k * 16, 16)] = sv * mult + cc
        pltpu.async_copy(table_h.at[gidx], rows_v, sem).wait()
        pltpu.sync_copy(rows_v, agg_sh.at[dst_v.at[j]], add=True)
        if want_deg and cc == 0:
          pltpu.sync_copy(ones_v, deg_sh.at[dst_v.at[j]], add=True)
        return carry
      lax.fori_loop(0, NCHUNK, chunk, 0)
      plsc.subcore_barrier()

      pltpu.sync_copy(agg_sh.at[pl.ds(sid * SL, SL)],
                      agg_out.at[cid, cc, pl.ds(sid * SL, SL)])
      if want_deg and cc == 0:
        pltpu.sync_copy(deg_sh.at[pl.ds(sid * SL, SL)],
                        deg_out.at[cid, pl.ds(sid * SL, SL)])
      if cc + 1 < n_pass:
        plsc.subcore_barrier()

  f = pl.kernel(
      body, out_type=out_type, mesh=mesh, scratch_types=scratch,
      compiler_params=pltpu.CompilerParams(use_tc_tiling_on_sc=False))
  res = f(table, src_r, dst_r)
  if want_deg:
    return res[0], res[1]
  return res[0]


def _rdeg(deg_ref):
  d = deg_ref[0, :, 0:1] + deg_ref[1, :, 0:1]
  return 1.0 / jnp.maximum(d, 1.0)


_P = jax.lax.Precision.HIGHEST


def _dot(a, b):
  return jnp.dot(a, b, preferred_element_type=jnp.float32, precision=_P)


def _tc1_body(nk, agg_ref, deg_ref, x_ref, wl_ref, wr_ref, b_ref, o_ref):
  acc = _dot(agg_ref[0, 0] + agg_ref[1, 0], wl_ref[0])
  for c in range(1, nk):
    acc = acc + _dot(agg_ref[0, c] + agg_ref[1, c], wl_ref[c])
  acc = acc * _rdeg(deg_ref) + _dot(x_ref[...], wr_ref[...])
  o_ref[...] = jnp.maximum(acc + b_ref[...], 0.0)


def _tc2_body(nk, agg_ref, deg_ref, h1_ref, wl_ref, wr_ref, b_ref, w3l_ref,
              w3r_ref, q_ref, r_ref):
  acc = _dot(agg_ref[0, 0] + agg_ref[1, 0], wl_ref[0])
  for c in range(1, nk):
    acc = acc + _dot(agg_ref[0, c] + agg_ref[1, c], wl_ref[c])
  acc = acc * _rdeg(deg_ref) + _dot(h1_ref[...], wr_ref[...])
  h2 = jnp.maximum(acc + b_ref[...], 0.0)
  q_ref[...] = _dot(h2, w3l_ref[...])
  r_ref[...] = _dot(h2, w3r_ref[...])


def _tc3_body(aggq_ref, deg_ref, r_ref, b_ref, o_ref):
  a = (aggq_ref[0, 0] + aggq_ref[1, 0]) * _rdeg(deg_ref)
  o_ref[...] = jnp.maximum(a + r_ref[...] + b_ref[...], 0.0)


def kernel(x, edge_index, batch, W1_l, W1_r, b1, W2_l, W2_r, b2,
           W3_l, W3_r, b3):
  f32 = jnp.float32
  src = edge_index[0]
  dst = edge_index[1]
  pad = EP - E_EDGES
  src_r = jnp.concatenate([src, jnp.zeros((pad,), jnp.int32)]).reshape(
      NW, NCHUNK, CH)
  dst_r = jnp.concatenate([dst, jnp.full((pad,), TRASH, jnp.int32)]).reshape(
      NW, NCHUNK, CH)
  xp = jnp.concatenate([x, jnp.zeros((NP - N_NODES, x.shape[1]), f32)])

  w3l16 = jnp.zeros((1024, 16), f32).at[:, :2].set(W3_l)
  w3r16 = jnp.zeros((1024, 16), f32).at[:, :2].set(W3_r)
  b1r = b1.reshape(1, 512)
  b2r = b2.reshape(1, 1024)
  b3r = jnp.zeros((1, 16), f32).at[0, :2].set(b3)

  aggA, deg = _sc_agg(xp.reshape(NP * 2, 64), src_r, dst_r, D=64,
                      n_pass=2, mult=2, want_deg=True)

  R = 512
  grid = (NP // R,)
  h1 = pl.pallas_call(
      functools.partial(_tc1_body, 2),
      grid=grid,
      in_specs=[
          pl.BlockSpec((2, 2, R, 64), lambda i: (0, 0, i, 0)),
          pl.BlockSpec((2, R, 16), lambda i: (0, i, 0)),
          pl.BlockSpec((R, 128), lambda i: (i, 0)),
          pl.BlockSpec((2, 64, 512), lambda i: (0, 0, 0)),
          pl.BlockSpec((128, 512), lambda i: (0, 0)),
          pl.BlockSpec((1, 512), lambda i: (0, 0)),
      ],
      out_specs=pl.BlockSpec((R, 512), lambda i: (i, 0)),
      out_shape=jax.ShapeDtypeStruct((NP, 512), f32),
  )(aggA, deg, xp, W1_l.reshape(2, 64, 512), W1_r, b1r)

  h1v = h1.reshape(NP * 8, 64)
  aggB = _sc_agg(h1v, src_r, dst_r, D=64, n_pass=8, mult=8, want_deg=False)

  q16, r16 = pl.pallas_call(
      functools.partial(_tc2_body, 8),
      grid=grid,
      in_specs=[
          pl.BlockSpec((2, 8, R, 64), lambda i: (0, 0, i, 0)),
          pl.BlockSpec((2, R, 16), lambda i: (0, i, 0)),
          pl.BlockSpec((R, 512), lambda i: (i, 0)),
          pl.BlockSpec((8, 64, 1024), lambda i: (0, 0, 0)),
          pl.BlockSpec((512, 1024), lambda i: (0, 0)),
          pl.BlockSpec((1, 1024), lambda i: (0, 0)),
          pl.BlockSpec((1024, 16), lambda i: (0, 0)),
          pl.BlockSpec((1024, 16), lambda i: (0, 0)),
      ],
      out_specs=[
          pl.BlockSpec((R, 16), lambda i: (i, 0)),
          pl.BlockSpec((R, 16), lambda i: (i, 0)),
      ],
      out_shape=[
          jax.ShapeDtypeStruct((NP, 16), f32),
          jax.ShapeDtypeStruct((NP, 16), f32),
      ],
  )(aggB, deg, h1, W2_l.reshape(8, 64, 1024), W2_r, b2r, w3l16, w3r16)

  aggQ = _sc_agg(q16, src_r, dst_r, D=16, n_pass=1, mult=1, want_deg=False)

  out = pl.pallas_call(
      _tc3_body,
      grid=(1,),
      in_specs=[
          pl.BlockSpec((2, 1, NP, 16), lambda i: (0, 0, 0, 0)),
          pl.BlockSpec((2, NP, 16), lambda i: (0, 0, 0)),
          pl.BlockSpec((NP, 16), lambda i: (0, 0)),
          pl.BlockSpec((1, 16), lambda i: (0, 0)),
      ],
      out_specs=pl.BlockSpec((NP, 16), lambda i: (0, 0)),
      out_shape=jax.ShapeDtypeStruct((NP, 16), f32),
  )(aggQ, deg, r16, b3r)

  return out[:N_NODES, :2]


# speedup vs baseline: 3.2258x; 1.1619x over previous
import functools

import jax
import jax.numpy as jnp
from jax import lax
from jax.experimental import pallas as pl
from jax.experimental.pallas import tpu as pltpu
from jax.experimental.pallas import tpu_sc as plsc

N_NODES = 10000
NP = 10240
E_EDGES = 160000
EP = 163840
NW = 32
EPW = EP // NW
CH = 128
NCHUNK = EPW // CH
SL = NP // 16
TRASH = NP - 1


def _sc_agg(table, src_r, dst_r, D, n_pass, mult, want_deg):
  mesh = plsc.VectorSubcoreMesh(core_axis_name="c", subcore_axis_name="s")
  out_type = [jax.ShapeDtypeStruct((2, n_pass, NP, D), jnp.float32)]
  if want_deg:
    out_type.append(jax.ShapeDtypeStruct((2, NP, 16), jnp.float32))

  NBUF = 4
  NGRP = NCHUNK // NBUF
  scratch = [
      pltpu.VMEM((NCHUNK, CH), jnp.int32),
      pltpu.VMEM((NCHUNK, CH), jnp.int32),
      pltpu.VMEM((NCHUNK, CH), jnp.int32),
      [pltpu.VMEM((CH, D), jnp.float32) for _ in range(NBUF)],
      pltpu.VMEM((CH, D), jnp.float32),
      pltpu.VMEM((CH, 16), jnp.float32),
      pltpu.VMEM((CH, 16), jnp.float32),
      pltpu.VMEM_SHARED((NP, D), jnp.float32),
      [pltpu.SemaphoreType.DMA for _ in range(NBUF)],
  ]
  if want_deg:
    scratch.append(pltpu.VMEM_SHARED((NP, 16), jnp.float32))

  def body(table_h, src_h, dst_h, *refs):
    if want_deg:
      agg_out, deg_out = refs[0], refs[1]
      refs = refs[2:]
    else:
      agg_out = refs[0]
      refs = refs[1:]
    if want_deg:
      (src_v, dst_v, gidx_all, rows, zbuf, ones_v, z16, agg_sh, sems,
       deg_sh) = refs
    else:
      (src_v, dst_v, gidx_all, rows, zbuf, ones_v, z16, agg_sh,
       sems) = refs
      deg_sh = None

    cid = lax.axis_index("c")
    sid = lax.axis_index("s")
    wid = cid * 16 + sid

    pltpu.sync_copy(src_h.at[wid], src_v)
    pltpu.sync_copy(dst_h.at[wid], dst_v)

    def fill(j, carry):
      for k in range(D // 16):
        zbuf[j, pl.ds(k * 16, 16)] = jnp.zeros((16,), jnp.float32)
      ones_v[j] = jnp.full((16,), 1.0, jnp.float32)
      z16[j] = jnp.zeros((16,), jnp.float32)
      return carry
    lax.fori_loop(0, CH, fill, 0)

    for cc in range(n_pass):
      for k in range(SL // CH):
        pltpu.sync_copy(zbuf, agg_sh.at[pl.ds(sid * SL + k * CH, CH)])
        if want_deg and cc == 0:
          pltpu.sync_copy(z16, deg_sh.at[pl.ds(sid * SL + k * CH, CH)])

      def gi(j, carry):
        for k in range(CH // 16):
          sv = src_v[j, pl.ds(k * 16, 16)]
          gidx_all[j, pl.ds(k * 16, 16)] = sv * mult + cc
        return carry
      lax.fori_loop(0, NCHUNK, gi, 0)
      plsc.subcore_barrier()

      for b in range(NBUF):
        pltpu.async_copy(table_h.at[gidx_all.at[b]], rows[b], sems[b])

      def group(g, carry):
        for b in range(NBUF):
          j = g * NBUF + b
          pltpu.make_async_copy(table_h.at[gidx_all.at[j]], rows[b],
                                sems[b]).wait()
          pltpu.sync_copy(rows[b], agg_sh.at[dst_v.at[j]], add=True)
          if want_deg and cc == 0:
            pltpu.sync_copy(ones_v, deg_sh.at[dst_v.at[j]], add=True)
          nj = j + NBUF

          @pl.when(nj < NCHUNK)
          def _():
            pltpu.async_copy(table_h.at[gidx_all.at[nj]], rows[b], sems[b])
        return carry
      lax.fori_loop(0, NGRP, group, 0)
      plsc.subcore_barrier()

      pltpu.sync_copy(agg_sh.at[pl.ds(sid * SL, SL)],
                      agg_out.at[cid, cc, pl.ds(sid * SL, SL)])
      if want_deg and cc == 0:
        pltpu.sync_copy(deg_sh.at[pl.ds(sid * SL, SL)],
                        deg_out.at[cid, pl.ds(sid * SL, SL)])
      if cc + 1 < n_pass:
        plsc.subcore_barrier()

  f = pl.kernel(
      body, out_type=out_type, mesh=mesh, scratch_types=scratch,
      compiler_params=pltpu.CompilerParams(use_tc_tiling_on_sc=False))
  res = f(table, src_r, dst_r)
  if want_deg:
    return res[0], res[1]
  return res[0]


def _rdeg(deg_ref):
  d = deg_ref[0, :, 0:1] + deg_ref[1, :, 0:1]
  return 1.0 / jnp.maximum(d, 1.0)


_P = jax.lax.Precision.HIGHEST


def _dot(a, b):
  return jnp.dot(a, b, preferred_element_type=jnp.float32, precision=_P)


def _tc1_body(nk, agg_ref, deg_ref, x_ref, wl_ref, wr_ref, b_ref, o_ref):
  acc = _dot(agg_ref[0, 0] + agg_ref[1, 0], wl_ref[0])
  for c in range(1, nk):
    acc = acc + _dot(agg_ref[0, c] + agg_ref[1, c], wl_ref[c])
  acc = acc * _rdeg(deg_ref) + _dot(x_ref[...], wr_ref[...])
  o_ref[...] = jnp.maximum(acc + b_ref[...], 0.0)


def _tc2_body(nk, agg_ref, deg_ref, h1_ref, wl_ref, wr_ref, b_ref, w3l_ref,
              w3r_ref, q_ref, r_ref):
  acc = _dot(agg_ref[0, 0] + agg_ref[1, 0], wl_ref[0])
  for c in range(1, nk):
    acc = acc + _dot(agg_ref[0, c] + agg_ref[1, c], wl_ref[c])
  acc = acc * _rdeg(deg_ref) + _dot(h1_ref[...], wr_ref[...])
  h2 = jnp.maximum(acc + b_ref[...], 0.0)
  q_ref[...] = _dot(h2, w3l_ref[...])
  r_ref[...] = _dot(h2, w3r_ref[...])


def _tc3_body(aggq_ref, deg_ref, r_ref, b_ref, o_ref):
  a = (aggq_ref[0, 0] + aggq_ref[1, 0]) * _rdeg(deg_ref)
  o_ref[...] = jnp.maximum(a + r_ref[...] + b_ref[...], 0.0)


def kernel(x, edge_index, batch, W1_l, W1_r, b1, W2_l, W2_r, b2,
           W3_l, W3_r, b3):
  f32 = jnp.float32
  src = edge_index[0]
  dst = edge_index[1]
  pad = EP - E_EDGES
  src_r = jnp.concatenate([src, jnp.zeros((pad,), jnp.int32)]).reshape(
      NW, NCHUNK, CH)
  dst_r = jnp.concatenate([dst, jnp.full((pad,), TRASH, jnp.int32)]).reshape(
      NW, NCHUNK, CH)
  xp = jnp.concatenate([x, jnp.zeros((NP - N_NODES, x.shape[1]), f32)])

  w3l16 = jnp.zeros((1024, 16), f32).at[:, :2].set(W3_l)
  w3r16 = jnp.zeros((1024, 16), f32).at[:, :2].set(W3_r)
  b1r = b1.reshape(1, 512)
  b2r = b2.reshape(1, 1024)
  b3r = jnp.zeros((1, 16), f32).at[0, :2].set(b3)

  aggA, deg = _sc_agg(xp.reshape(NP * 2, 64), src_r, dst_r, D=64,
                      n_pass=2, mult=2, want_deg=True)

  R = 512
  grid = (NP // R,)
  h1 = pl.pallas_call(
      functools.partial(_tc1_body, 2),
      grid=grid,
      in_specs=[
          pl.BlockSpec((2, 2, R, 64), lambda i: (0, 0, i, 0)),
          pl.BlockSpec((2, R, 16), lambda i: (0, i, 0)),
          pl.BlockSpec((R, 128), lambda i: (i, 0)),
          pl.BlockSpec((2, 64, 512), lambda i: (0, 0, 0)),
          pl.BlockSpec((128, 512), lambda i: (0, 0)),
          pl.BlockSpec((1, 512), lambda i: (0, 0)),
      ],
      out_specs=pl.BlockSpec((R, 512), lambda i: (i, 0)),
      out_shape=jax.ShapeDtypeStruct((NP, 512), f32),
  )(aggA, deg, xp, W1_l.reshape(2, 64, 512), W1_r, b1r)

  h1v = h1.reshape(NP * 8, 64)
  aggB = _sc_agg(h1v, src_r, dst_r, D=64, n_pass=8, mult=8, want_deg=False)

  q16, r16 = pl.pallas_call(
      functools.partial(_tc2_body, 8),
      grid=grid,
      in_specs=[
          pl.BlockSpec((2, 8, R, 64), lambda i: (0, 0, i, 0)),
          pl.BlockSpec((2, R, 16), lambda i: (0, i, 0)),
          pl.BlockSpec((R, 512), lambda i: (i, 0)),
          pl.BlockSpec((8, 64, 1024), lambda i: (0, 0, 0)),
          pl.BlockSpec((512, 1024), lambda i: (0, 0)),
          pl.BlockSpec((1, 1024), lambda i: (0, 0)),
          pl.BlockSpec((1024, 16), lambda i: (0, 0)),
          pl.BlockSpec((1024, 16), lambda i: (0, 0)),
      ],
      out_specs=[
          pl.BlockSpec((R, 16), lambda i: (i, 0)),
          pl.BlockSpec((R, 16), lambda i: (i, 0)),
      ],
      out_shape=[
          jax.ShapeDtypeStruct((NP, 16), f32),
          jax.ShapeDtypeStruct((NP, 16), f32),
      ],
  )(aggB, deg, h1, W2_l.reshape(8, 64, 1024), W2_r, b2r, w3l16, w3r16)

  aggQ = _sc_agg(q16, src_r, dst_r, D=16, n_pass=1, mult=1, want_deg=False)

  out = pl.pallas_call(
      _tc3_body,
      grid=(1,),
      in_specs=[
          pl.BlockSpec((2, 1, NP, 16), lambda i: (0, 0, 0, 0)),
          pl.BlockSpec((2, NP, 16), lambda i: (0, 0, 0)),
          pl.BlockSpec((NP, 16), lambda i: (0, 0)),
          pl.BlockSpec((1, 16), lambda i: (0, 0)),
      ],
      out_specs=pl.BlockSpec((NP, 16), lambda i: (0, 0)),
      out_shape=jax.ShapeDtypeStruct((NP, 16), f32),
  )(aggQ, deg, r16, b3r)

  return out[:N_NODES, :2]


# speedup vs baseline: 3.7987x; 1.1776x over previous
import functools

import jax
import jax.numpy as jnp
from jax import lax
from jax.experimental import pallas as pl
from jax.experimental.pallas import tpu as pltpu
from jax.experimental.pallas import tpu_sc as plsc

N_NODES = 10000
NP = 10240
E_EDGES = 160000
EP = 163840
NW = 32
EPW = EP // NW
CH = 128
NCHUNK = EPW // CH
SL = NP // 16
TRASH = NP - 1


def _sc_agg(table, src_r, dst_r, D, n_pass, mult, want_deg):
  mesh = plsc.VectorSubcoreMesh(core_axis_name="c", subcore_axis_name="s")
  out_type = [jax.ShapeDtypeStruct((2, n_pass, NP, D), jnp.float32)]
  if want_deg:
    out_type.append(jax.ShapeDtypeStruct((2, NP, 16), jnp.float32))

  NBUF = 4
  NGRP = NCHUNK // NBUF
  scratch = [
      pltpu.VMEM((NCHUNK, CH), jnp.int32),
      pltpu.VMEM((NCHUNK, CH), jnp.int32),
      pltpu.VMEM((NCHUNK, CH), jnp.int32),
      [pltpu.VMEM((CH, D), jnp.float32) for _ in range(NBUF)],
      pltpu.VMEM((CH, D), jnp.float32),
      pltpu.VMEM((CH, 16), jnp.float32),
      pltpu.VMEM((CH, 16), jnp.float32),
      pltpu.VMEM_SHARED((NP, D), jnp.float32),
      [pltpu.SemaphoreType.DMA for _ in range(NBUF)],
      [pltpu.SemaphoreType.DMA for _ in range(NBUF)],
  ]
  if want_deg:
    scratch.append(pltpu.VMEM_SHARED((NP, 16), jnp.float32))

  def body(table_h, src_h, dst_h, *refs):
    if want_deg:
      agg_out, deg_out = refs[0], refs[1]
      refs = refs[2:]
    else:
      agg_out = refs[0]
      refs = refs[1:]
    if want_deg:
      (src_v, dst_v, gidx_all, rows, zbuf, ones_v, z16, agg_sh, sems,
       ssems, deg_sh) = refs
    else:
      (src_v, dst_v, gidx_all, rows, zbuf, ones_v, z16, agg_sh, sems,
       ssems) = refs
      deg_sh = None

    cid = lax.axis_index("c")
    sid = lax.axis_index("s")
    wid = cid * 16 + sid

    pltpu.sync_copy(src_h.at[wid], src_v)
    pltpu.sync_copy(dst_h.at[wid], dst_v)

    def fill(j, carry):
      for k in range(D // 16):
        zbuf[j, pl.ds(k * 16, 16)] = jnp.zeros((16,), jnp.float32)
      ones_v[j] = jnp.full((16,), 1.0, jnp.float32)
      z16[j] = jnp.zeros((16,), jnp.float32)
      return carry
    lax.fori_loop(0, CH, fill, 0)

    for cc in range(n_pass):
      for k in range(SL // CH):
        pltpu.sync_copy(zbuf, agg_sh.at[pl.ds(sid * SL + k * CH, CH)])
        if want_deg and cc == 0:
          pltpu.sync_copy(z16, deg_sh.at[pl.ds(sid * SL + k * CH, CH)])

      def gi(j, carry):
        for k in range(CH // 16):
          sv = src_v[j, pl.ds(k * 16, 16)]
          gidx_all[j, pl.ds(k * 16, 16)] = sv * mult + cc
        return carry
      lax.fori_loop(0, NCHUNK, gi, 0)
      plsc.subcore_barrier()

      pltpu.async_copy(table_h.at[gidx_all.at[0]], rows[0], sems[0])

      def group(g, carry):
        for b in range(NBUF):
          j = g * NBUF + b
          bn = (b + 1) % NBUF

          @pl.when(j >= NBUF - 1)
          def _():
            pltpu.make_async_copy(rows[bn], agg_sh.at[dst_v.at[0]],
                                  ssems[bn]).wait()

          @pl.when(j + 1 < NCHUNK)
          def _():
            pltpu.async_copy(table_h.at[gidx_all.at[j + 1]], rows[bn],
                             sems[bn])
          pltpu.make_async_copy(table_h.at[gidx_all.at[j]], rows[b],
                                sems[b]).wait()
          pltpu.async_copy(rows[b], agg_sh.at[dst_v.at[j]], ssems[b],
                           add=True)
          if want_deg and cc == 0:
            pltpu.sync_copy(ones_v, deg_sh.at[dst_v.at[j]], add=True)
        return carry
      lax.fori_loop(0, NGRP, group, 0)
      for b in range(1, NBUF):
        j = NCHUNK - NBUF + b
        pltpu.make_async_copy(rows[j % NBUF], agg_sh.at[dst_v.at[0]],
                              ssems[j % NBUF]).wait()
      plsc.subcore_barrier()

      pltpu.sync_copy(agg_sh.at[pl.ds(sid * SL, SL)],
                      agg_out.at[cid, cc, pl.ds(sid * SL, SL)])
      if want_deg and cc == 0:
        pltpu.sync_copy(deg_sh.at[pl.ds(sid * SL, SL)],
                        deg_out.at[cid, pl.ds(sid * SL, SL)])
      if cc + 1 < n_pass:
        plsc.subcore_barrier()

  f = pl.kernel(
      body, out_type=out_type, mesh=mesh, scratch_types=scratch,
      compiler_params=pltpu.CompilerParams(use_tc_tiling_on_sc=False))
  res = f(table, src_r, dst_r)
  if want_deg:
    return res[0], res[1]
  return res[0]


def _rdeg(deg_ref):
  d = deg_ref[0, :, 0:1] + deg_ref[1, :, 0:1]
  return 1.0 / jnp.maximum(d, 1.0)


def _dot(a, b):
  return jnp.dot(a.astype(jnp.bfloat16), b.astype(jnp.bfloat16),
                 preferred_element_type=jnp.float32)


def _tc1_body(nk, agg_ref, deg_ref, x_ref, wl_ref, wr_ref, b_ref, o_ref):
  rd = _rdeg(deg_ref)
  acc = _dot((agg_ref[0, 0] + agg_ref[1, 0]) * rd, wl_ref[0])
  for c in range(1, nk):
    acc = acc + _dot((agg_ref[0, c] + agg_ref[1, c]) * rd, wl_ref[c])
  acc = acc + _dot(x_ref[...], wr_ref[...])
  o_ref[...] = jnp.maximum(acc + b_ref[...], 0.0)


def _tc2_body(nk, agg_ref, deg_ref, h1_ref, wl_ref, wr_ref, b_ref, w3l_ref,
              w3r_ref, q_ref, r_ref):
  rd = _rdeg(deg_ref)
  acc = _dot((agg_ref[0, 0] + agg_ref[1, 0]) * rd, wl_ref[0])
  for c in range(1, nk):
    acc = acc + _dot((agg_ref[0, c] + agg_ref[1, c]) * rd, wl_ref[c])
  acc = acc + _dot(h1_ref[...], wr_ref[...])
  h2 = jnp.maximum(acc + b_ref[...], 0.0)
  q_ref[...] = jnp.dot(h2, w3l_ref[...], preferred_element_type=jnp.float32,
                       precision=jax.lax.Precision.HIGHEST)
  r_ref[...] = _dot(h2, w3r_ref[...])


def _tc3_body(aggq_ref, deg_ref, r_ref, b_ref, o_ref):
  a = (aggq_ref[0, 0] + aggq_ref[1, 0]) * _rdeg(deg_ref)
  o_ref[...] = jnp.maximum(a + r_ref[...] + b_ref[...], 0.0)


def kernel(x, edge_index, batch, W1_l, W1_r, b1, W2_l, W2_r, b2,
           W3_l, W3_r, b3):
  f32 = jnp.float32
  src = edge_index[0]
  dst = edge_index[1]
  pad = EP - E_EDGES
  src_r = jnp.concatenate([src, jnp.zeros((pad,), jnp.int32)]).reshape(
      NW, NCHUNK, CH)
  dst_r = jnp.concatenate([dst, jnp.full((pad,), TRASH, jnp.int32)]).reshape(
      NW, NCHUNK, CH)
  xp = jnp.concatenate([x, jnp.zeros((NP - N_NODES, x.shape[1]), f32)])

  w3l16 = jnp.zeros((1024, 16), f32).at[:, :2].set(W3_l)
  w3r16 = jnp.zeros((1024, 16), f32).at[:, :2].set(W3_r)
  b1r = b1.reshape(1, 512)
  b2r = b2.reshape(1, 1024)
  b3r = jnp.zeros((1, 16), f32).at[0, :2].set(b3)

  aggA, deg = _sc_agg(xp.reshape(NP * 2, 64), src_r, dst_r, D=64,
                      n_pass=2, mult=2, want_deg=True)

  R = 512
  grid = (NP // R,)
  h1 = pl.pallas_call(
      functools.partial(_tc1_body, 2),
      grid=grid,
      in_specs=[
          pl.BlockSpec((2, 2, R, 64), lambda i: (0, 0, i, 0)),
          pl.BlockSpec((2, R, 16), lambda i: (0, i, 0)),
          pl.BlockSpec((R, 128), lambda i: (i, 0)),
          pl.BlockSpec((2, 64, 512), lambda i: (0, 0, 0)),
          pl.BlockSpec((128, 512), lambda i: (0, 0)),
          pl.BlockSpec((1, 512), lambda i: (0, 0)),
      ],
      out_specs=pl.BlockSpec((R, 512), lambda i: (i, 0)),
      out_shape=jax.ShapeDtypeStruct((NP, 512), f32),
  )(aggA, deg, xp, W1_l.reshape(2, 64, 512), W1_r, b1r)

  h1v = h1.reshape(NP * 8, 64)
  aggB = _sc_agg(h1v, src_r, dst_r, D=64, n_pass=8, mult=8, want_deg=False)

  q16, r16 = pl.pallas_call(
      functools.partial(_tc2_body, 8),
      grid=grid,
      in_specs=[
          pl.BlockSpec((2, 8, R, 64), lambda i: (0, 0, i, 0)),
          pl.BlockSpec((2, R, 16), lambda i: (0, i, 0)),
          pl.BlockSpec((R, 512), lambda i: (i, 0)),
          pl.BlockSpec((8, 64, 1024), lambda i: (0, 0, 0)),
          pl.BlockSpec((512, 1024), lambda i: (0, 0)),
          pl.BlockSpec((1, 1024), lambda i: (0, 0)),
          pl.BlockSpec((1024, 16), lambda i: (0, 0)),
          pl.BlockSpec((1024, 16), lambda i: (0, 0)),
      ],
      out_specs=[
          pl.BlockSpec((R, 16), lambda i: (i, 0)),
          pl.BlockSpec((R, 16), lambda i: (i, 0)),
      ],
      out_shape=[
          jax.ShapeDtypeStruct((NP, 16), f32),
          jax.ShapeDtypeStruct((NP, 16), f32),
      ],
  )(aggB, deg, h1, W2_l.reshape(8, 64, 1024), W2_r, b2r, w3l16, w3r16)

  aggQ = _sc_agg(q16, src_r, dst_r, D=16, n_pass=1, mult=1, want_deg=False)

  out = pl.pallas_call(
      _tc3_body,
      grid=(1,),
      in_specs=[
          pl.BlockSpec((2, 1, NP, 16), lambda i: (0, 0, 0, 0)),
          pl.BlockSpec((2, NP, 16), lambda i: (0, 0, 0)),
          pl.BlockSpec((NP, 16), lambda i: (0, 0)),
          pl.BlockSpec((1, 16), lambda i: (0, 0)),
      ],
      out_specs=pl.BlockSpec((NP, 16), lambda i: (0, 0)),
      out_shape=jax.ShapeDtypeStruct((NP, 16), f32),
  )(aggQ, deg, r16, b3r)

  return out[:N_NODES, :2]
